# Initial kernel scaffold; baseline (speedup 1.0000x reference)
import jax
import jax.numpy as jnp
from jax.experimental import pallas as pl


def kernel(x, weight):
    raise NotImplementedError("write your pallas kernel here")


# speedup vs baseline: 1.8736x; 1.8736x over previous
import functools

import jax
import jax.numpy as jnp
from jax import lax
from jax.experimental import pallas as pl
from jax.experimental.pallas import tpu as pltpu
from jax.experimental.pallas import tpu_sc as plsc

_NC = 2
_NS = 16
_NW = _NC * _NS


@functools.cache
def _build_lookup(n_rows: int, d: int, chunk: int):
  per_w = n_rows // _NW
  assert per_w % chunk == 0 and (per_w * _NW) == n_rows
  n_chunks = per_w // chunk
  assert n_chunks % 2 == 0

  mesh = plsc.VectorSubcoreMesh(
      core_axis_name="c", subcore_axis_name="s",
      num_cores=_NC, num_subcores=_NS)

  @functools.partial(
      pl.kernel,
      mesh=mesh,
      out_type=jax.ShapeDtypeStruct((n_rows, d), jnp.float32),
      compiler_params=pltpu.CompilerParams(use_tc_tiling_on_sc=False),
      scratch_types=[
          pltpu.VMEM((per_w,), jnp.int32),
          pltpu.VMEM((2, chunk, d), jnp.float32),
          pltpu.SemaphoreType.DMA,
          pltpu.SemaphoreType.DMA,
          pltpu.SemaphoreType.DMA,
          pltpu.SemaphoreType.DMA,
      ],
  )
  def lookup(x_hbm, w_hbm, out_hbm, idx_v, rows_v, g0, g1, o0, o1):
    wid = lax.axis_index("s") * _NC + lax.axis_index("c")
    base = wid * per_w
    pltpu.sync_copy(x_hbm.at[pl.ds(base, per_w)], idx_v)

    gsems = (g0, g1)
    osems = (o0, o1)

    def gather(c, b):
      return pltpu.make_async_copy(
          w_hbm.at[idx_v.at[pl.ds(c * chunk, chunk)]],
          rows_v.at[b], gsems[b])

    def put(c, b):
      return pltpu.make_async_copy(
          rows_v.at[b],
          out_hbm.at[pl.ds(base + c * chunk, chunk)], osems[b])

    gather(0, 0).start()

    @pl.loop(0, n_chunks, step=2)
    def _(c0):
      for b in range(2):
        c = c0 + b
        nxt = 1 - b

        @pl.when(c + 1 < n_chunks)
        def _():
          @pl.when(c >= 1)
          def _():
            put(c - 1, nxt).wait()
          gather(c + 1, nxt).start()

        gather(c, b).wait()
        put(c, b).start()

    put(n_chunks - 2, 0).wait()
    put(n_chunks - 1, 1).wait()

  return lookup


def kernel(x, weight):
  b, h = x.shape
  d = weight.shape[1]
  flat = x.reshape(b * h).astype(jnp.int32)
  out = _build_lookup(b * h, d, 512)(flat, weight)
  return out.reshape(b, h, d)
